# Initial kernel scaffold; baseline (speedup 1.0000x reference)
import jax
import jax.numpy as jnp
from jax.experimental import pallas as pl


def kernel(neighbor_index, vertices, directions):
    raise NotImplementedError("write your pallas kernel here")


# speedup vs baseline: 106.9095x; 106.9095x over previous
import functools

import jax
import jax.numpy as jnp
from jax import lax
from jax.experimental import pallas as pl
from jax.experimental.pallas import tpu as pltpu
from jax.experimental.pallas import tpu_sc as plsc

BS = 4
V = 10000
VP = 10240
NN = 16
K = 32
CHUNK = VP // 8
GROUPS = CHUNK // 16


def _rsqrt(nsq):
    i = lax.bitcast_convert_type(nsq, jnp.int32)
    i = jnp.int32(0x5F3759DF) - lax.shift_right_logical(i, jnp.int32(1))
    y = lax.bitcast_convert_type(i, jnp.float32)
    halfx = nsq * jnp.float32(0.5)
    for _ in range(3):
        y = y * (jnp.float32(1.5) - halfx * y * y)
    return y


def _body(idx_hbm, vx_hbm, vy_hbm, vz_hbm, w_hbm, out_hbm,
          vx_v, vy_v, vz_v, idx_v, out_v, w_v):
    c = lax.axis_index("c")
    s = lax.axis_index("s")
    wid = s * 2 + c
    b = wid // 8
    base = (wid % 8) * CHUNK

    pltpu.sync_copy(vx_hbm.at[b], vx_v)
    pltpu.sync_copy(vy_hbm.at[b], vy_v)
    pltpu.sync_copy(vz_hbm.at[b], vz_v)
    pltpu.sync_copy(idx_hbm.at[b, :, pl.ds(base, CHUNK)], idx_v)
    pltpu.sync_copy(w_hbm, w_v)

    for h in range(2):
        sl = pl.ds(h * 16, 16)
        r0 = w_v[0, sl]
        r1 = w_v[1, sl]
        r2 = w_v[2, sl]
        inv = _rsqrt(r0 * r0 + r1 * r1 + r2 * r2)
        w_v[0, sl] = r0 * inv
        w_v[1, sl] = r1 * inv
        w_v[2, sl] = r2 * inv

    w_rows = [[w_v[d, pl.ds(h * 16, 16)] for h in range(2)] for d in range(3)]

    def group(g, carry):
        vsl = pl.ds(g * 16, 16)
        gsl = pl.ds(base + g * 16, 16)
        selx = vx_v[gsl]
        sely = vy_v[gsl]
        selz = vz_v[gsl]
        sdx, sdy, sdz = [], [], []
        for j in range(NN):
            ij = idx_v[j, vsl]
            gx = plsc.load_gather(vx_v, [ij])
            gy = plsc.load_gather(vy_v, [ij])
            gz = plsc.load_gather(vz_v, [ij])
            dx = gx - selx
            dy = gy - sely
            dz = gz - selz
            inv = _rsqrt(dx * dx + dy * dy + dz * dz)
            sdx.append(dx * inv)
            sdy.append(dy * inv)
            sdz.append(dz * inv)
        for k in range(K):
            kh, kl = k // 16, k % 16
            w0 = jnp.broadcast_to(w_rows[0][kh][kl], (16,))
            w1 = jnp.broadcast_to(w_rows[1][kh][kl], (16,))
            w2 = jnp.broadcast_to(w_rows[2][kh][kl], (16,))
            acc = sdx[0] * w0 + sdy[0] * w1 + sdz[0] * w2
            for j in range(1, NN):
                acc = jnp.maximum(acc, sdx[j] * w0 + sdy[j] * w1 + sdz[j] * w2)
            out_v[k, vsl] = jnp.maximum(acc, jnp.float32(0.0))
        return carry

    lax.fori_loop(0, GROUPS, group, 0)
    pltpu.sync_copy(out_v, out_hbm.at[b, :, pl.ds(base, CHUNK)])


def kernel(neighbor_index, vertices, directions):
    ni = neighbor_index.astype(jnp.int32)
    idx_t = jnp.pad(jnp.transpose(ni, (0, 2, 1)),
                    ((0, 0), (0, 0), (0, VP - V)))
    vpad = jnp.pad(vertices, ((0, 0), (0, VP - V), (0, 0)))
    vx = vpad[:, :, 0]
    vy = vpad[:, :, 1]
    vz = vpad[:, :, 2]

    mesh = plsc.VectorSubcoreMesh(core_axis_name="c", subcore_axis_name="s")
    f = pl.kernel(
        _body,
        mesh=mesh,
        out_type=jax.ShapeDtypeStruct((BS, K, VP), jnp.float32),
        compiler_params=pltpu.CompilerParams(needs_layout_passes=False),
        scratch_types=[
            pltpu.VMEM((VP,), jnp.float32),
            pltpu.VMEM((VP,), jnp.float32),
            pltpu.VMEM((VP,), jnp.float32),
            pltpu.VMEM((NN, CHUNK), jnp.int32),
            pltpu.VMEM((K, CHUNK), jnp.float32),
            pltpu.VMEM((3, K), jnp.float32),
        ],
    )
    out_t = f(idx_t, vx, vy, vz, directions.astype(jnp.float32))
    return jnp.transpose(out_t, (0, 2, 1))[:, :V, :]


# speedup vs baseline: 120.5995x; 1.1281x over previous
import functools

import jax
import jax.numpy as jnp
from jax import lax
from jax.experimental import pallas as pl
from jax.experimental.pallas import tpu as pltpu
from jax.experimental.pallas import tpu_sc as plsc

BS = 4
V = 10000
VP = 10240
NN = 16
K = 32
CHUNK = VP // 8
GROUPS = CHUNK // 16


def _rsqrt(nsq):
    i = lax.bitcast_convert_type(nsq, jnp.int32)
    i = jnp.int32(0x5F3759DF) - lax.shift_right_logical(i, jnp.int32(1))
    y = lax.bitcast_convert_type(i, jnp.float32)
    halfx = nsq * jnp.float32(0.5)
    for _ in range(2):
        y = y * (jnp.float32(1.5) - halfx * y * y)
    return y


def _tree_max(xs):
    xs = list(xs)
    while len(xs) > 1:
        nxt = [jnp.maximum(a, b) for a, b in zip(xs[::2], xs[1::2])]
        if len(xs) % 2:
            nxt.append(xs[-1])
        xs = nxt
    return xs[0]


def _body(idx_hbm, vx_hbm, vy_hbm, vz_hbm, w_hbm, out_hbm,
          vx_v, vy_v, vz_v, idx_v, out_v, w_v, acc_v):
    c = lax.axis_index("c")
    s = lax.axis_index("s")
    wid = s * 2 + c
    b = wid // 8
    base = (wid % 8) * CHUNK

    pltpu.sync_copy(vx_hbm.at[b], vx_v)
    pltpu.sync_copy(vy_hbm.at[b], vy_v)
    pltpu.sync_copy(vz_hbm.at[b], vz_v)
    pltpu.sync_copy(idx_hbm.at[b, :, pl.ds(base, CHUNK)], idx_v)
    pltpu.sync_copy(w_hbm, w_v)

    for h in range(2):
        sl = pl.ds(h * 16, 16)
        r0 = w_v[0, sl]
        r1 = w_v[1, sl]
        r2 = w_v[2, sl]
        inv = _rsqrt(r0 * r0 + r1 * r1 + r2 * r2)
        w_v[0, sl] = r0 * inv
        w_v[1, sl] = r1 * inv
        w_v[2, sl] = r2 * inv

    w_rows = [[w_v[d, pl.ds(h * 16, 16)] for h in range(2)] for d in range(3)]

    def group(g, carry):
        vsl = pl.ds(g * 16, 16)
        gsl = pl.ds(base + g * 16, 16)
        selx = vx_v[gsl]
        sely = vy_v[gsl]
        selz = vz_v[gsl]
        for half in range(2):
            sd = []
            for j in range(half * 8, half * 8 + 8):
                ij = idx_v[j, vsl]
                gx = plsc.load_gather(vx_v, [ij])
                gy = plsc.load_gather(vy_v, [ij])
                gz = plsc.load_gather(vz_v, [ij])
                dx = gx - selx
                dy = gy - sely
                dz = gz - selz
                inv = _rsqrt(dx * dx + dy * dy + dz * dz)
                sd.append((dx * inv, dy * inv, dz * inv))
            for k in range(K):
                kh, kl = k // 16, k % 16
                w0 = jnp.broadcast_to(w_rows[0][kh][kl], (16,))
                w1 = jnp.broadcast_to(w_rows[1][kh][kl], (16,))
                w2 = jnp.broadcast_to(w_rows[2][kh][kl], (16,))
                m = _tree_max(
                    [x * w0 + y * w1 + z * w2 for (x, y, z) in sd])
                if half == 0:
                    acc_v[k] = m
                else:
                    out_v[k, vsl] = jnp.maximum(
                        jnp.maximum(acc_v[k], m), jnp.float32(0.0))
        return carry

    lax.fori_loop(0, GROUPS, group, 0)
    pltpu.sync_copy(out_v, out_hbm.at[b, :, pl.ds(base, CHUNK)])


def kernel(neighbor_index, vertices, directions):
    ni = neighbor_index.astype(jnp.int32)
    idx_t = jnp.pad(jnp.transpose(ni, (0, 2, 1)),
                    ((0, 0), (0, 0), (0, VP - V)))
    vpad = jnp.pad(vertices, ((0, 0), (0, VP - V), (0, 0)))
    vx = vpad[:, :, 0]
    vy = vpad[:, :, 1]
    vz = vpad[:, :, 2]

    mesh = plsc.VectorSubcoreMesh(core_axis_name="c", subcore_axis_name="s")
    f = pl.kernel(
        _body,
        mesh=mesh,
        out_type=jax.ShapeDtypeStruct((BS, K, VP), jnp.float32),
        compiler_params=pltpu.CompilerParams(needs_layout_passes=False),
        scratch_types=[
            pltpu.VMEM((VP,), jnp.float32),
            pltpu.VMEM((VP,), jnp.float32),
            pltpu.VMEM((VP,), jnp.float32),
            pltpu.VMEM((NN, CHUNK), jnp.int32),
            pltpu.VMEM((K, CHUNK), jnp.float32),
            pltpu.VMEM((3, K), jnp.float32),
            pltpu.VMEM((K, 16), jnp.float32),
        ],
    )
    out_t = f(idx_t, vx, vy, vz, directions.astype(jnp.float32))
    return jnp.transpose(out_t, (0, 2, 1))[:, :V, :]


# speedup vs baseline: 123.5162x; 1.0242x over previous
import jax
import jax.numpy as jnp
from jax import lax
from jax.experimental import pallas as pl
from jax.experimental.pallas import tpu as pltpu
from jax.experimental.pallas import tpu_sc as plsc

BS = 4
V = 10000
VP = 10240
NN = 16
K = 32
CHUNK = VP // 8
GROUPS = CHUNK // 16


def _rsqrt(nsq):
    i = lax.bitcast_convert_type(nsq, jnp.int32)
    i = jnp.int32(0x5F3759DF) - lax.shift_right_logical(i, jnp.int32(1))
    y = lax.bitcast_convert_type(i, jnp.float32)
    halfx = nsq * jnp.float32(0.5)
    for _ in range(2):
        y = y * (jnp.float32(1.5) - halfx * y * y)
    return y


def _tree_max(xs):
    xs = list(xs)
    while len(xs) > 1:
        nxt = [jnp.maximum(a, b) for a, b in zip(xs[::2], xs[1::2])]
        if len(xs) % 2:
            nxt.append(xs[-1])
        xs = nxt
    return xs[0]


def _body(idx_hbm, vx_hbm, vy_hbm, vz_hbm, w_hbm, out_hbm,
          vx_v, vy_v, vz_v, idx_v, out_v, w_v, acc_v,
          sem_i, sem_x, sem_y, sem_z, sem_w):
    c = lax.axis_index("c")
    s = lax.axis_index("s")
    wid = s * 2 + c
    b = wid // 8
    base = (wid % 8) * CHUNK

    cp_i = pltpu.async_copy(idx_hbm.at[b, :, pl.ds(base, CHUNK)], idx_v, sem_i)
    cp_x = pltpu.async_copy(vx_hbm.at[b], vx_v, sem_x)
    cp_y = pltpu.async_copy(vy_hbm.at[b], vy_v, sem_y)
    cp_z = pltpu.async_copy(vz_hbm.at[b], vz_v, sem_z)
    cp_w = pltpu.async_copy(w_hbm, w_v, sem_w)

    cp_w.wait()
    for h in range(2):
        sl = pl.ds(h * 16, 16)
        r0 = w_v[0, sl]
        r1 = w_v[1, sl]
        r2 = w_v[2, sl]
        inv = _rsqrt(r0 * r0 + r1 * r1 + r2 * r2)
        w_v[0, sl] = r0 * inv
        w_v[1, sl] = r1 * inv
        w_v[2, sl] = r2 * inv

    w_rows = [[w_v[d, pl.ds(h * 16, 16)] for h in range(2)] for d in range(3)]

    cp_x.wait()
    cp_y.wait()
    cp_z.wait()
    cp_i.wait()

    def group(g, carry):
        vsl = pl.ds(g * 16, 16)
        gsl = pl.ds(base + g * 16, 16)
        selx = vx_v[gsl]
        sely = vy_v[gsl]
        selz = vz_v[gsl]
        for half in range(2):
            sd = []
            for j in range(half * 8, half * 8 + 8):
                ij = idx_v[j, vsl]
                gx = plsc.load_gather(vx_v, [ij])
                gy = plsc.load_gather(vy_v, [ij])
                gz = plsc.load_gather(vz_v, [ij])
                dx = gx - selx
                dy = gy - sely
                dz = gz - selz
                inv = _rsqrt(dx * dx + dy * dy + dz * dz)
                sd.append((dx * inv, dy * inv, dz * inv))
            for k in range(K):
                kh, kl = k // 16, k % 16
                w0 = jnp.broadcast_to(w_rows[0][kh][kl], (16,))
                w1 = jnp.broadcast_to(w_rows[1][kh][kl], (16,))
                w2 = jnp.broadcast_to(w_rows[2][kh][kl], (16,))
                m = _tree_max(
                    [x * w0 + y * w1 + z * w2 for (x, y, z) in sd])
                if half == 0:
                    acc_v[k] = m
                else:
                    out_v[k, vsl] = jnp.maximum(
                        jnp.maximum(acc_v[k], m), jnp.float32(0.0))
        return carry

    lax.fori_loop(0, GROUPS, group, 0)
    pltpu.sync_copy(out_v, out_hbm.at[b, :, pl.ds(base, CHUNK)])


def kernel(neighbor_index, vertices, directions):
    ni = neighbor_index.astype(jnp.int32)
    idx_t = jnp.pad(jnp.transpose(ni, (0, 2, 1)),
                    ((0, 0), (0, 0), (0, VP - V)))
    vpad = jnp.pad(vertices, ((0, 0), (0, VP - V), (0, 0)))
    vx = vpad[:, :, 0]
    vy = vpad[:, :, 1]
    vz = vpad[:, :, 2]

    mesh = plsc.VectorSubcoreMesh(core_axis_name="c", subcore_axis_name="s")
    f = pl.kernel(
        _body,
        mesh=mesh,
        out_type=jax.ShapeDtypeStruct((BS, K, VP), jnp.float32),
        compiler_params=pltpu.CompilerParams(needs_layout_passes=False),
        scratch_types=[
            pltpu.VMEM((VP,), jnp.float32),
            pltpu.VMEM((VP,), jnp.float32),
            pltpu.VMEM((VP,), jnp.float32),
            pltpu.VMEM((NN, CHUNK), jnp.int32),
            pltpu.VMEM((K, CHUNK), jnp.float32),
            pltpu.VMEM((3, K), jnp.float32),
            pltpu.VMEM((K, 16), jnp.float32),
            pltpu.SemaphoreType.DMA,
            pltpu.SemaphoreType.DMA,
            pltpu.SemaphoreType.DMA,
            pltpu.SemaphoreType.DMA,
            pltpu.SemaphoreType.DMA,
        ],
    )
    out_t = f(idx_t, vx, vy, vz, directions.astype(jnp.float32))
    return jnp.transpose(out_t, (0, 2, 1))[:, :V, :]
